# baseline (speedup 1.0000x reference)
import jax
import jax.numpy as jnp
import numpy as np
from jax.experimental import pallas as pl

_ANCHORS = np.array([[10.0, 13.0], [16.0, 30.0], [33.0, 23.0]], dtype=np.float32)
_IMG_SIZE = 608.0


def _decode_kernel(x_ref, bbox_ref, conf_ref, cls_ref, *, stride, anchors):
    a = pl.program_id(1)
    p = x_ref[0]
    G = p.shape[1]
    gx = jax.lax.broadcasted_iota(jnp.int32, (G, G), 1).astype(p.dtype)
    gy = jax.lax.broadcasted_iota(jnp.int32, (G, G), 0).astype(p.dtype)
    aw = jnp.where(a == 0, anchors[0, 0], jnp.where(a == 1, anchors[1, 0], anchors[2, 0]))
    ah = jnp.where(a == 0, anchors[0, 1], jnp.where(a == 1, anchors[1, 1], anchors[2, 1]))
    bx = (jax.nn.sigmoid(p[0]) + gx) * stride
    by = (jax.nn.sigmoid(p[1]) + gy) * stride
    bw = jnp.exp(p[2]) * aw
    bh = jnp.exp(p[3]) * ah
    bbox = jnp.stack((bx, by, bw, bh), axis=0)
    bbox_ref[0, 0] = jnp.transpose(bbox, (1, 2, 0))
    conf_ref[0, 0] = jax.nn.sigmoid(p[4])
    cls_ref[0, 0] = jnp.transpose(jax.nn.sigmoid(p[5:]), (1, 2, 0))


def kernel(x):
    B = x.shape[0]
    G = x.shape[2]
    A = _ANCHORS.shape[0]
    C = x.shape[1] // A - 5
    stride = _IMG_SIZE / G

    bbox, conf, cls_ = pl.pallas_call(
        lambda *refs: _decode_kernel(*refs, stride=stride, anchors=_ANCHORS),
        grid=(B, A),
        in_specs=[
            pl.BlockSpec((1, C + 5, G, G), lambda b, a: (b, a, 0, 0)),
        ],
        out_specs=[
            pl.BlockSpec((1, 1, G, G, 4), lambda b, a: (b, a, 0, 0, 0)),
            pl.BlockSpec((1, 1, G, G), lambda b, a: (b, a, 0, 0)),
            pl.BlockSpec((1, 1, G, G, C), lambda b, a: (b, a, 0, 0, 0)),
        ],
        out_shape=[
            jax.ShapeDtypeStruct((B, A, G, G, 4), x.dtype),
            jax.ShapeDtypeStruct((B, A, G, G), x.dtype),
            jax.ShapeDtypeStruct((B, A, G, G, C), x.dtype),
        ],
    )(x)

    return (bbox, conf, cls_)
